# baseline (speedup 1.0000x reference)
import functools

import jax
import jax.numpy as jnp
from jax.experimental import pallas as pl
from jax.experimental.pallas import tpu as pltpu


def _fused_kernel(xv_ref, x4d_ref, w1t_ref, w2t_ref, bcv_ref, wfc_ref, bfc_ref,
                  xout_ref, y_ref, x1_ref, x3_ref, cpsem,
                  *, hid, out_dim, inv_hw, inv_ohw):
    b = pl.program_id(0)
    nb = pl.num_programs(0)

    passthrough = pltpu.make_async_copy(x4d_ref, xout_ref, cpsem)

    @pl.when(b == 0)
    def _():
        passthrough.start()

    x48 = xv_ref[0]
    C = x48.shape[0]
    hw = x48.shape[1] * x48.shape[2]
    xb = x48.reshape(C, hw)

    b1 = bcv_ref[0:hid, :]
    b2 = bcv_ref[hid:hid + out_dim, :]

    h = jnp.dot(w1t_ref[...], xb, preferred_element_type=jnp.float32) + b1
    h = jnp.maximum(h, 0.0)
    y = jnp.dot(w2t_ref[...], h.astype(jnp.bfloat16),
                preferred_element_type=jnp.float32) + b2
    y_ref[0] = y

    ones = jnp.ones((hw, 1), jnp.bfloat16)
    pooled = jnp.dot(xb, ones,
                     preferred_element_type=jnp.float32) * inv_hw
    wfc1 = wfc_ref[0:C, :]
    wfc2t = wfc_ref[C:, :]
    hfc = jax.lax.dot_general(
        pooled, wfc1, (((0,), (0,)), ((), ())),
        preferred_element_type=jnp.float32)
    hfc = jnp.maximum(hfc + bfc_ref[:, 0:hid], 0.0)
    x1 = jax.lax.dot_general(
        hfc, wfc2t, (((1,), (1,)), ((), ())),
        preferred_element_type=jnp.float32)
    x1_ref[0] = x1 + bfc_ref[:, hid:hid + out_dim]
    x3_ref[0] = (jnp.sum(y) * inv_ohw).reshape(1, 1)

    @pl.when(b == nb - 1)
    def _():
        passthrough.wait()


def kernel(x, w1_fc, b1_fc, w2_fc, b2_fc, w1_cv, b1_cv, w2_cv, b2_cv):
    B, C, H, W = x.shape
    HW = H * W
    S = HW // 128
    hid = w1_cv.shape[1]
    out_dim = w2_cv.shape[1]

    xv = x.reshape(B, C, S, 128).astype(jnp.bfloat16)
    w1t = w1_cv.T.astype(jnp.bfloat16)
    w2t = w2_cv.T.astype(jnp.bfloat16)
    bcv = jnp.concatenate([b1_cv, b2_cv]).reshape(hid + out_dim, 1)
    wfc = jnp.concatenate([w1_fc, w2_fc.T], axis=0)
    bfc = jnp.concatenate([b1_fc, b2_fc]).reshape(1, hid + out_dim)

    body = functools.partial(_fused_kernel,
                             hid=hid, out_dim=out_dim,
                             inv_hw=1.0 / HW,
                             inv_ohw=1.0 / (out_dim * HW))

    xout, y, x1o, x3o = pl.pallas_call(
        body,
        grid=(B,),
        in_specs=[
            pl.BlockSpec((1, C, S, 128), lambda b: (b, 0, 0, 0)),
            pl.BlockSpec(memory_space=pl.ANY),
            pl.BlockSpec((hid, C), lambda b: (0, 0)),
            pl.BlockSpec((out_dim, hid), lambda b: (0, 0)),
            pl.BlockSpec((hid + out_dim, 1), lambda b: (0, 0)),
            pl.BlockSpec((C + out_dim, hid), lambda b: (0, 0)),
            pl.BlockSpec((1, hid + out_dim), lambda b: (0, 0)),
        ],
        out_specs=[
            pl.BlockSpec(memory_space=pl.ANY),
            pl.BlockSpec((1, out_dim, HW), lambda b: (b, 0, 0)),
            pl.BlockSpec((1, 1, out_dim), lambda b: (b, 0, 0)),
            pl.BlockSpec((1, 1, 1), lambda b: (b, 0, 0)),
        ],
        out_shape=[
            jax.ShapeDtypeStruct((B, C, H, W), jnp.float32),
            jax.ShapeDtypeStruct((B, out_dim, HW), jnp.float32),
            jax.ShapeDtypeStruct((B, 1, out_dim), jnp.float32),
            jax.ShapeDtypeStruct((B, 1, 1), jnp.float32),
        ],
        scratch_shapes=[
            pltpu.SemaphoreType.DMA,
        ],
        compiler_params=pltpu.CompilerParams(
            dimension_semantics=("arbitrary",)),
    )(xv, x, w1t, w2t, bcv, wfc, bfc)

    x1 = x1o[:, 0, :]
    x3 = x3o[:, :, 0]
    return xout, x1, y, x3


# speedup vs baseline: 15.0847x; 15.0847x over previous
import functools

import jax
import jax.numpy as jnp
from jax.experimental import pallas as pl
from jax.experimental.pallas import tpu as pltpu


def _fused_kernel(xv_ref, x4d_ref, w1t_ref, w2t_ref, bcv_ref, wfc_ref, bfc_ref,
                  xout_ref, y_ref, x1_ref, x3_ref,
                  *, hid, out_dim, inv_hw, inv_ohw):
    del x4d_ref, xout_ref
    x48 = xv_ref[0]
    C = x48.shape[0]
    hw = x48.shape[1] * x48.shape[2]
    xb = x48.reshape(C, hw)

    b1 = bcv_ref[0:hid, :]
    b2 = bcv_ref[hid:hid + out_dim, :]

    h = jnp.dot(w1t_ref[...], xb, preferred_element_type=jnp.float32) + b1
    h = jnp.maximum(h, 0.0)
    y = jnp.dot(w2t_ref[...], h.astype(jnp.bfloat16),
                preferred_element_type=jnp.float32) + b2
    y_ref[0] = y

    ones = jnp.ones((hw, 1), jnp.bfloat16)
    pooled = jnp.dot(xb, ones,
                     preferred_element_type=jnp.float32) * inv_hw
    wfc1 = wfc_ref[0:C, :]
    wfc2t = wfc_ref[C:, :]
    hfc = jax.lax.dot_general(
        pooled, wfc1, (((0,), (0,)), ((), ())),
        preferred_element_type=jnp.float32)
    hfc = jnp.maximum(hfc + bfc_ref[:, 0:hid], 0.0)
    x1 = jax.lax.dot_general(
        hfc, wfc2t, (((1,), (1,)), ((), ())),
        preferred_element_type=jnp.float32)
    x1_ref[0] = x1 + bfc_ref[:, hid:hid + out_dim]
    x3_ref[0] = (jnp.sum(y) * inv_ohw).reshape(1, 1)


def kernel(x, w1_fc, b1_fc, w2_fc, b2_fc, w1_cv, b1_cv, w2_cv, b2_cv):
    B, C, H, W = x.shape
    HW = H * W
    S = HW // 128
    hid = w1_cv.shape[1]
    out_dim = w2_cv.shape[1]

    xv = x.reshape(B, C, S, 128).astype(jnp.bfloat16)
    w1t = w1_cv.T.astype(jnp.bfloat16)
    w2t = w2_cv.T.astype(jnp.bfloat16)
    bcv = jnp.concatenate([b1_cv, b2_cv]).reshape(hid + out_dim, 1)
    wfc = jnp.concatenate([w1_fc, w2_fc.T], axis=0)
    bfc = jnp.concatenate([b1_fc, b2_fc]).reshape(1, hid + out_dim)

    body = functools.partial(_fused_kernel,
                             hid=hid, out_dim=out_dim,
                             inv_hw=1.0 / HW,
                             inv_ohw=1.0 / (out_dim * HW))

    xout, y, x1o, x3o = pl.pallas_call(
        body,
        grid=(B,),
        in_specs=[
            pl.BlockSpec((1, C, S, 128), lambda b: (b, 0, 0, 0)),
            pl.BlockSpec(memory_space=pl.ANY),
            pl.BlockSpec((hid, C), lambda b: (0, 0)),
            pl.BlockSpec((out_dim, hid), lambda b: (0, 0)),
            pl.BlockSpec((hid + out_dim, 1), lambda b: (0, 0)),
            pl.BlockSpec((C + out_dim, hid), lambda b: (0, 0)),
            pl.BlockSpec((1, hid + out_dim), lambda b: (0, 0)),
        ],
        out_specs=[
            pl.BlockSpec(memory_space=pl.ANY),
            pl.BlockSpec((1, out_dim, HW), lambda b: (b, 0, 0)),
            pl.BlockSpec((1, 1, out_dim), lambda b: (b, 0, 0)),
            pl.BlockSpec((1, 1, 1), lambda b: (b, 0, 0)),
        ],
        out_shape=[
            jax.ShapeDtypeStruct((B, C, H, W), jnp.float32),
            jax.ShapeDtypeStruct((B, out_dim, HW), jnp.float32),
            jax.ShapeDtypeStruct((B, 1, out_dim), jnp.float32),
            jax.ShapeDtypeStruct((B, 1, 1), jnp.float32),
        ],
        input_output_aliases={1: 0},
        compiler_params=pltpu.CompilerParams(
            dimension_semantics=("arbitrary",)),
    )(xv, x, w1t, w2t, bcv, wfc, bfc)

    x1 = x1o[:, 0, :]
    x3 = x3o[:, :, 0]
    return xout, x1, y, x3


# speedup vs baseline: 44.2424x; 2.9329x over previous
import functools

import jax
import jax.numpy as jnp
from jax.experimental import pallas as pl
from jax.experimental.pallas import tpu as pltpu


def _fused_kernel(x_ref, w1t_ref, b1_ref, w2t_ref, b2_ref,
                  wfc1_ref, bfc1_ref, wfc2_ref, bfc2_ref,
                  y_ref, x1_ref, x3_ref, xacc_ref, yacc_ref,
                  *, inv_hw, inv_ohw):
    t = pl.program_id(1)
    nt = pl.num_programs(1)

    x = x_ref[0]

    h = jnp.dot(w1t_ref[...], x.astype(jnp.bfloat16),
                preferred_element_type=jnp.float32) + b1_ref[...]
    h = jnp.maximum(h, 0.0)
    y = jnp.dot(w2t_ref[...], h.astype(jnp.bfloat16),
                preferred_element_type=jnp.float32) + b2_ref[...]
    y_ref[0] = y

    xpart = jnp.sum(x, axis=-1).reshape(1, -1)
    ypart = jnp.sum(y).reshape(1, 1)

    @pl.when(t == 0)
    def _():
        xacc_ref[...] = xpart
        yacc_ref[...] = ypart

    @pl.when(t > 0)
    def _():
        xacc_ref[...] += xpart
        yacc_ref[...] += ypart

    @pl.when(t == nt - 1)
    def _():
        pooled = xacc_ref[...] * inv_hw
        hfc = jnp.dot(pooled, wfc1_ref[...],
                      preferred_element_type=jnp.float32) + bfc1_ref[...]
        hfc = jnp.maximum(hfc, 0.0)
        x1 = jnp.dot(hfc, wfc2_ref[...],
                     preferred_element_type=jnp.float32) + bfc2_ref[...]
        x1_ref[0] = x1
        x3_ref[0] = yacc_ref[...] * inv_ohw


def _pick_tile_hw(hw):
    for t in (1024, 512, 256, 128):
        if hw % t == 0:
            return t
    return hw


def kernel(x, w1_fc, b1_fc, w2_fc, b2_fc, w1_cv, b1_cv, w2_cv, b2_cv):
    B, C, H, W = x.shape
    HW = H * W
    hid = w1_cv.shape[1]
    out_dim = w2_cv.shape[1]

    tile_hw = _pick_tile_hw(HW)
    n_tiles = HW // tile_hw

    x_bcl = x.reshape(B, C, HW)
    w1t = w1_cv.T.astype(jnp.bfloat16)
    w2t = w2_cv.T.astype(jnp.bfloat16)
    b1c = b1_cv.reshape(hid, 1)
    b2c = b2_cv.reshape(out_dim, 1)
    bfc1 = b1_fc.reshape(1, hid)
    bfc2 = b2_fc.reshape(1, out_dim)

    body = functools.partial(_fused_kernel,
                             inv_hw=1.0 / HW,
                             inv_ohw=1.0 / (out_dim * HW))

    y, x1o, x3o = pl.pallas_call(
        body,
        grid=(B, n_tiles),
        in_specs=[
            pl.BlockSpec((1, C, tile_hw), lambda b, t: (b, 0, t)),
            pl.BlockSpec((hid, C), lambda b, t: (0, 0)),
            pl.BlockSpec((hid, 1), lambda b, t: (0, 0)),
            pl.BlockSpec((out_dim, hid), lambda b, t: (0, 0)),
            pl.BlockSpec((out_dim, 1), lambda b, t: (0, 0)),
            pl.BlockSpec((C, hid), lambda b, t: (0, 0)),
            pl.BlockSpec((1, hid), lambda b, t: (0, 0)),
            pl.BlockSpec((hid, out_dim), lambda b, t: (0, 0)),
            pl.BlockSpec((1, out_dim), lambda b, t: (0, 0)),
        ],
        out_specs=[
            pl.BlockSpec((1, out_dim, tile_hw), lambda b, t: (b, 0, t)),
            pl.BlockSpec((1, 1, out_dim), lambda b, t: (b, 0, 0)),
            pl.BlockSpec((1, 1, 1), lambda b, t: (b, 0, 0)),
        ],
        out_shape=[
            jax.ShapeDtypeStruct((B, out_dim, HW), jnp.float32),
            jax.ShapeDtypeStruct((B, 1, out_dim), jnp.float32),
            jax.ShapeDtypeStruct((B, 1, 1), jnp.float32),
        ],
        scratch_shapes=[
            pltpu.VMEM((1, C), jnp.float32),
            pltpu.VMEM((1, 1), jnp.float32),
        ],
        compiler_params=pltpu.CompilerParams(
            dimension_semantics=("parallel", "arbitrary")),
    )(x_bcl, w1t, b1c, w2t, b2c, w1_fc, bfc1, w2_fc, bfc2)

    x1 = x1o[:, 0, :]
    x3 = x3o[:, :, 0]
    return x, x1, y, x3
